# Initial kernel scaffold; baseline (speedup 1.0000x reference)
import jax
import jax.numpy as jnp
from jax.experimental import pallas as pl


def kernel(edge_index, h_val_init, W_v2c, W_c2v, W_v2v, W_ih, W_hh, b_ih, b_hh, w_pol, steps):
    raise NotImplementedError("write your pallas kernel here")


# speedup vs baseline: 2.9792x; 2.9792x over previous
import jax
import jax.numpy as jnp
from jax import lax
from jax.experimental import pallas as pl
from jax.experimental.pallas import tpu as pltpu
from jax.experimental.pallas import tpu_sc as plsc

N_NODES = 10000
H = 128
CHUNK = 128
NC = 2
NS = 16
NP = -(-(N_NODES + 1) // (NS * CHUNK)) * (NS * CHUNK)
BR = 2048


def _sc_segment_sum(idx_g, idx_s, tab, zeros):
    rt = idx_s.shape[0]
    cpt = rt // (NC * NS)
    blk = cpt // 2
    stripe = NP // NS
    sub = stripe // CHUNK

    def body(gidx_h, sidx_h, tab_h, zeros_h, out_h,
             gi_v, si_v, r0, r1, s0, s1, acc):
        c = lax.axis_index("c")
        s = lax.axis_index("s")
        base = (c * NS + s) * cpt
        pltpu.sync_copy(zeros_h.at[pl.ds(s * stripe, stripe)],
                        acc.at[pl.ds(s * stripe, stripe)])
        plsc.subcore_barrier()

        for b in range(2):
            pltpu.sync_copy(gidx_h.at[pl.ds(base + b * blk, blk)], gi_v)
            pltpu.sync_copy(sidx_h.at[pl.ds(base + b * blk, blk)], si_v)

            pltpu.async_copy(tab_h.at[gi_v.at[0]], r0, s0)
            pltpu.async_copy(tab_h.at[gi_v.at[1]], r1, s1)

            def pair(k, carry):
                j0 = 2 * k
                pltpu.make_async_copy(tab_h.at[gi_v.at[j0]], r0, s0).wait()
                pltpu.sync_copy(r0, acc.at[si_v.at[j0]], add=True)

                @pl.when(j0 + 2 < blk)
                def _():
                    pltpu.async_copy(tab_h.at[gi_v.at[j0 + 2]], r0, s0)

                j1 = j0 + 1
                pltpu.make_async_copy(tab_h.at[gi_v.at[j1]], r1, s1).wait()
                pltpu.sync_copy(r1, acc.at[si_v.at[j1]], add=True)

                @pl.when(j1 + 2 < blk)
                def _():
                    pltpu.async_copy(tab_h.at[gi_v.at[j1 + 2]], r1, s1)

                return carry

            lax.fori_loop(0, blk // 2, pair, 0)
        plsc.subcore_barrier()

        for t in range(sub):
            off = s * stripe + t * CHUNK
            pltpu.sync_copy(acc.at[pl.ds(off, CHUNK)], r0)
            pltpu.sync_copy(r0, out_h.at[pl.ds(c * NP + off, CHUNK)])

    mesh = plsc.VectorSubcoreMesh(core_axis_name="c", subcore_axis_name="s",
                                  num_cores=NC, num_subcores=NS)
    return pl.kernel(
        body,
        out_type=jax.ShapeDtypeStruct((NC * NP, H), jnp.float32),
        mesh=mesh,
        scratch_types=[
            pltpu.VMEM((blk, CHUNK), jnp.int32),
            pltpu.VMEM((blk, CHUNK), jnp.int32),
            pltpu.VMEM((CHUNK, H), jnp.float32),
            pltpu.VMEM((CHUNK, H), jnp.float32),
            pltpu.SemaphoreType.DMA,
            pltpu.SemaphoreType.DMA,
            pltpu.VMEM_SHARED((NP, H), jnp.float32),
        ],
    )(idx_g, idx_s, tab, zeros)


def _tc_mm(tab, w):
    def body(t_ref, w_ref, o_ref):
        o_ref[...] = jnp.dot(t_ref[...], w_ref[...],
                             preferred_element_type=jnp.float32)

    return pl.pallas_call(
        body,
        grid=(NP // BR,),
        in_specs=[
            pl.BlockSpec((BR, H), lambda i: (i, 0)),
            pl.BlockSpec((H, H), lambda i: (0, 0)),
        ],
        out_specs=pl.BlockSpec((BR, H), lambda i: (i, 0)),
        out_shape=jax.ShapeDtypeStruct((NP, H), jnp.float32),
    )(tab, w)


def _tc_relu_mm(part, w):
    part3 = part.reshape(NC, NP, H)

    def body(p_ref, w_ref, o_ref):
        r = jnp.maximum(p_ref[0] + p_ref[1], 0.0)
        o_ref[...] = jnp.dot(r, w_ref[...],
                             preferred_element_type=jnp.float32)

    return pl.pallas_call(
        body,
        grid=(NP // BR,),
        in_specs=[
            pl.BlockSpec((NC, BR, H), lambda i: (0, i, 0)),
            pl.BlockSpec((H, H), lambda i: (0, 0)),
        ],
        out_specs=pl.BlockSpec((BR, H), lambda i: (i, 0)),
        out_shape=jax.ShapeDtypeStruct((NP, H), jnp.float32),
    )(part3, w)


def _tc_update(part, h, w_v2c, w_v2v, w_ih, w_hh, b_ih, b_hh):
    part3 = part.reshape(NC, NP, H)

    def body(p_ref, h_ref, wc_ref, wv_ref, wih_ref, whh_ref,
             bih_ref, bhh_ref, hn_ref, m1_ref):
        y = jnp.maximum(p_ref[0] + p_ref[1], 0.0)
        z = jnp.maximum(
            jnp.dot(y, wv_ref[...], preferred_element_type=jnp.float32), 0.0)
        hh = h_ref[...]
        gi = lax.dot_general(z, wih_ref[...], (((1,), (1,)), ((), ())),
                             preferred_element_type=jnp.float32) + bih_ref[...]
        gh = lax.dot_general(hh, whh_ref[...], (((1,), (1,)), ((), ())),
                             preferred_element_type=jnp.float32) + bhh_ref[...]
        r = jax.nn.sigmoid(gi[:, :H] + gh[:, :H])
        zg = jax.nn.sigmoid(gi[:, H:2 * H] + gh[:, H:2 * H])
        n = jnp.tanh(gi[:, 2 * H:] + r * gh[:, 2 * H:])
        hn = (1.0 - zg) * n + zg * hh
        row0 = pl.program_id(0) * BR
        mask = row0 + lax.broadcasted_iota(jnp.int32, (BR, 1), 0) < N_NODES
        hn = jnp.where(mask, hn, 0.0)
        hn_ref[...] = hn
        m1_ref[...] = jnp.dot(hn, wc_ref[...],
                              preferred_element_type=jnp.float32)

    wspec = pl.BlockSpec((3 * H, H), lambda i: (0, 0))
    bspec = pl.BlockSpec((3 * H,), lambda i: (0,))
    return pl.pallas_call(
        body,
        grid=(NP // BR,),
        in_specs=[
            pl.BlockSpec((NC, BR, H), lambda i: (0, i, 0)),
            pl.BlockSpec((BR, H), lambda i: (i, 0)),
            pl.BlockSpec((H, H), lambda i: (0, 0)),
            pl.BlockSpec((H, H), lambda i: (0, 0)),
            wspec, wspec, bspec, bspec,
        ],
        out_specs=(pl.BlockSpec((BR, H), lambda i: (i, 0)),
                   pl.BlockSpec((BR, H), lambda i: (i, 0))),
        out_shape=(jax.ShapeDtypeStruct((NP, H), jnp.float32),
                   jax.ShapeDtypeStruct((NP, H), jnp.float32)),
    )(part3, h, w_v2c, w_v2v, w_ih, w_hh, b_ih, b_hh)


def _tc_policy(h, w_pol):
    def body(h_ref, wp_ref, o_ref):
        lg = jnp.dot(h_ref[...], wp_ref[...][:, None],
                     preferred_element_type=jnp.float32)
        o_ref[...] = jnp.broadcast_to(lg, (NP, H))

    return pl.pallas_call(
        body,
        out_shape=jax.ShapeDtypeStruct((NP, H), jnp.float32),
    )(h, w_pol)


def kernel(edge_index, h_val_init, W_v2c, W_c2v, W_v2v, W_ih, W_hh,
           b_ih, b_hh, w_pol, steps):
    e = edge_index.shape[1]
    rt = -(-e // (CHUNK * NC * NS * 8)) * (NC * NS * 8)
    pe = rt * CHUNK

    src = edge_index[0].astype(jnp.int32)
    dst = edge_index[1].astype(jnp.int32)
    fill = jnp.full((pe - e,), N_NODES, jnp.int32)
    src_p = jnp.concatenate([src, fill]).reshape(rt, CHUNK)
    dst_p = jnp.concatenate([dst, fill]).reshape(rt, CHUNK)

    zeros = jnp.zeros((NP, H), jnp.float32)
    hrow = jnp.broadcast_to(h_val_init, (N_NODES, H))
    pad = jnp.zeros((NP - N_NODES, H), jnp.float32)
    h0 = jnp.concatenate([hrow, pad])
    m1_0 = _tc_mm(h0, W_v2c)

    def step(_, carry):
        h, m1 = carry
        part1 = _sc_segment_sum(src_p, dst_p, m1, zeros)
        m2 = _tc_relu_mm(part1, W_c2v)
        part2 = _sc_segment_sum(dst_p, src_p, m2, zeros)
        return _tc_update(part2, h, W_v2c, W_v2v, W_ih, W_hh, b_ih, b_hh)

    h_fin, _m1 = lax.fori_loop(0, steps, step, (h0, m1_0))
    lg = _tc_policy(h_fin, w_pol)
    return lg[:N_NODES, 0]
